# baseline (speedup 1.0000x reference)
import jax
import jax.numpy as jnp
from jax import lax
from jax.experimental import pallas as pl
from jax.experimental.pallas import tpu as pltpu
from jax.experimental.pallas import tpu_sc as plsc

N = 10000
E = 160000
D = 256
G = 128
HALF = 128

NSUB = 16
CHUNK = 125
EROWS = E // CHUNK
TROWS = EROWS // NSUB
IGRP = 40
TROWS_D = EROWS // (2 * NSUB)
STRIPE = 624
TAILOFF = STRIPE * NSUB
TAIL = N - TAILOFF

_f32 = jnp.float32


def _mesh():
  return plsc.VectorSubcoreMesh(core_axis_name="c", subcore_axis_name="s")


def _stripe_copy(s, src_ref, dst_ref):
  base = pl.multiple_of(s * STRIPE, 8)
  pltpu.sync_copy(src_ref.at[pl.ds(base, STRIPE)], dst_ref.at[pl.ds(base, STRIPE)])

  @pl.when(s == 0)
  def _():
    pltpu.sync_copy(src_ref.at[pl.ds(TAILOFF, TAIL)],
                    dst_ref.at[pl.ds(TAILOFF, TAIL)])


def _prop_body(u0_hbm, u1_hbm, src_hbm, dst_hbm, out0_hbm, out1_hbm,
               acc_sh, sidx_v, didx_v, rows_a, rows_b, sem_a, sem_b):
  c = lax.axis_index("c")
  s = lax.axis_index("s")
  ib = pl.multiple_of(s * TROWS, 8)

  def per_core(fn):
    @pl.when(c == 0)
    def _():
      fn(u0_hbm, out0_hbm)

    @pl.when(c == 1)
    def _():
      fn(u1_hbm, out1_hbm)

  per_core(lambda u_hbm, out_hbm: _stripe_copy(s, u_hbm, acc_sh))
  plsc.subcore_barrier()

  def gather_start(k, rows_v, sem):
    per_core(lambda u_hbm, out_hbm:
             pltpu.async_copy(u_hbm.at[sidx_v.at[k]], rows_v, sem))

  def gather_wait(k, rows_v, sem):
    per_core(lambda u_hbm, out_hbm:
             pltpu.make_async_copy(u_hbm.at[sidx_v.at[k]], rows_v, sem).wait())

  @pl.loop(0, TROWS, step=IGRP)
  def _(jg):
    off = pl.multiple_of(ib + jg, 8)
    pltpu.sync_copy(src_hbm.at[pl.ds(off, IGRP)], sidx_v)
    pltpu.sync_copy(dst_hbm.at[pl.ds(off, IGRP)], didx_v)
    gather_start(0, rows_a, sem_a)

    @pl.loop(0, IGRP, step=2)
    def _(k):
      gather_wait(k, rows_a, sem_a)
      gather_start(k + 1, rows_b, sem_b)
      pltpu.sync_copy(rows_a, acc_sh.at[didx_v.at[k]], add=True)
      gather_wait(k + 1, rows_b, sem_b)

      @pl.when(k + 2 < IGRP)
      def _():
        gather_start(k + 2, rows_a, sem_a)

      pltpu.sync_copy(rows_b, acc_sh.at[didx_v.at[k + 1]], add=True)

  plsc.subcore_barrier()
  per_core(lambda u_hbm, out_hbm: _stripe_copy(s, acc_sh, out_hbm))


@jax.jit
def _sc_propagate(u0, u1, src2d, dst2d):
  run = pl.kernel(
      _prop_body,
      out_type=(jax.ShapeDtypeStruct((N, HALF), _f32),
                jax.ShapeDtypeStruct((N, HALF), _f32)),
      mesh=_mesh(),
      scratch_types=[
          pltpu.VMEM_SHARED((N, HALF), _f32),
          pltpu.VMEM((IGRP, CHUNK), jnp.int32),
          pltpu.VMEM((IGRP, CHUNK), jnp.int32),
          pltpu.VMEM((CHUNK, HALF), _f32),
          pltpu.VMEM((CHUNK, HALF), _f32),
          pltpu.SemaphoreType.DMA,
          pltpu.SemaphoreType.DMA,
      ],
  )
  return run(u0, u1, src2d, dst2d)


def _deg_body(dst_hbm, zeros_hbm, deg0_hbm, deg1_hbm, acc_sh, didx_v, ones_v):
  c = lax.axis_index("c")
  s = lax.axis_index("s")
  ib = pl.multiple_of(c * (EROWS // 2) + s * TROWS_D, 8)
  pltpu.sync_copy(dst_hbm.at[pl.ds(ib, TROWS_D)], didx_v)

  @pl.loop(0, CHUNK)
  def _(r):
    @pl.loop(0, HALF, step=16)
    def _(q):
      ones_v[r, pl.ds(q, 16)] = jnp.full((16,), 1.0, _f32)

  _stripe_copy(s, zeros_hbm, acc_sh)
  plsc.subcore_barrier()

  @pl.loop(0, TROWS_D)
  def _(j):
    pltpu.sync_copy(ones_v, acc_sh.at[didx_v.at[j]], add=True)

  plsc.subcore_barrier()

  @pl.when(c == 0)
  def _():
    _stripe_copy(s, acc_sh, deg0_hbm)

  @pl.when(c == 1)
  def _():
    _stripe_copy(s, acc_sh, deg1_hbm)


@jax.jit
def _sc_degree(dst2d, zeros_half):
  run = pl.kernel(
      _deg_body,
      out_type=(jax.ShapeDtypeStruct((N, HALF), _f32),
                jax.ShapeDtypeStruct((N, HALF), _f32)),
      mesh=_mesh(),
      scratch_types=[
          pltpu.VMEM_SHARED((N, HALF), _f32),
          pltpu.VMEM((TROWS_D, CHUNK), jnp.int32),
          pltpu.VMEM((CHUNK, HALF), _f32),
      ],
  )
  return run(dst2d, zeros_half)


ROWB = 1000


def _tc_first_body(x_ref, d_ref, d2_ref, w_ref, u0_ref, u1_ref, a_ref):
  a = lax.rsqrt(d_ref[:, 0:1] + d2_ref[:, 0:1] + 1.0)
  a_ref[...] = a
  u = a * jnp.dot(x_ref[...].astype(jnp.bfloat16),
                  w_ref[...].astype(jnp.bfloat16),
                  preferred_element_type=_f32)
  u0_ref[...] = u[:, :HALF]
  u1_ref[...] = u[:, HALF:]


def _tc_mid_body(s0_ref, s1_ref, a_ref, b_ref, w_ref, u0_ref, u1_ref):
  a = a_ref[...]
  h = jnp.concatenate([s0_ref[...], s1_ref[...]], axis=1)
  h = jnp.maximum(a * h + b_ref[...], 0.0)
  u = a * jnp.dot(h.astype(jnp.bfloat16), w_ref[...].astype(jnp.bfloat16),
                  preferred_element_type=_f32)
  u0_ref[...] = u[:, :HALF]
  u1_ref[...] = u[:, HALF:]


def _tc_final_body(s0_ref, s1_ref, a_ref, b_ref, batch_ref,
                   wl1_ref, bl1_ref, wl2_ref, bl2_ref, out_ref,
                   pooled_sc, cnt_sc):
  i = pl.program_id(0)

  @pl.when(i == 0)
  def _():
    pooled_sc[...] = jnp.zeros_like(pooled_sc)
    cnt_sc[...] = jnp.zeros_like(cnt_sc)

  a = a_ref[...]
  h = jnp.concatenate([s0_ref[...], s1_ref[...]], axis=1)
  h = jnp.maximum(a * h + b_ref[...], 0.0)
  g = batch_ref[0, 0]
  oh = (g[:, None] == lax.broadcasted_iota(jnp.int32, (ROWB, G), 1))
  oh = oh.astype(_f32)
  pooled_sc[...] += lax.dot_general(
      oh, h, (((0,), (0,)), ((), ())), preferred_element_type=_f32)
  cnt_sc[...] += lax.dot_general(
      oh, jnp.ones((ROWB, 1), _f32), (((0,), (0,)), ((), ())),
      preferred_element_type=_f32)

  @pl.when(i == pl.num_programs(0) - 1)
  def _():
    pooled = pooled_sc[...] / jnp.maximum(cnt_sc[...], 1.0)
    p1 = jnp.maximum(
        jnp.dot(pooled, wl1_ref[...], preferred_element_type=_f32)
        + bl1_ref[...], 0.0)
    out_ref[...] = (jnp.dot(p1, wl2_ref[...], preferred_element_type=_f32)
                    + bl2_ref[...])


def _row_specs():
  return [
      pl.BlockSpec((ROWB, HALF), lambda i: (i, 0)),
      pl.BlockSpec((ROWB, HALF), lambda i: (i, 0)),
      pl.BlockSpec((ROWB, 1), lambda i: (i, 0)),
      pl.BlockSpec((1, D), lambda i: (0, 0)),
  ]


@jax.jit
def _tc_first(x, deg0, deg1, W):
  return pl.pallas_call(
      _tc_first_body,
      grid=(N // ROWB,),
      in_specs=[
          pl.BlockSpec((ROWB, D), lambda i: (i, 0)),
          pl.BlockSpec((ROWB, HALF), lambda i: (i, 0)),
          pl.BlockSpec((ROWB, HALF), lambda i: (i, 0)),
          pl.BlockSpec((D, D), lambda i: (0, 0)),
      ],
      out_specs=[pl.BlockSpec((ROWB, HALF), lambda i: (i, 0))] * 2
      + [pl.BlockSpec((ROWB, 1), lambda i: (i, 0))],
      out_shape=[jax.ShapeDtypeStruct((N, HALF), _f32)] * 2
      + [jax.ShapeDtypeStruct((N, 1), _f32)],
  )(x, deg0, deg1, W)


@jax.jit
def _tc_mid(s0, s1, a2d, b2d, W):
  return pl.pallas_call(
      _tc_mid_body,
      grid=(N // ROWB,),
      in_specs=_row_specs() + [pl.BlockSpec((D, D), lambda i: (0, 0))],
      out_specs=[pl.BlockSpec((ROWB, HALF), lambda i: (i, 0))] * 2,
      out_shape=[jax.ShapeDtypeStruct((N, HALF), _f32)] * 2,
  )(s0, s1, a2d, b2d, W)


@jax.jit
def _tc_final(s0, s1, a2d, b2d, batch3d, Wl1, bl1_2d, Wl2, bl2_2d):
  return pl.pallas_call(
      _tc_final_body,
      grid=(N // ROWB,),
      in_specs=_row_specs() + [
          pl.BlockSpec((1, 1, ROWB), lambda i: (i, 0, 0)),
          pl.BlockSpec((D, HALF), lambda i: (0, 0)),
          pl.BlockSpec((1, HALF), lambda i: (0, 0)),
          pl.BlockSpec((HALF, HALF), lambda i: (0, 0)),
          pl.BlockSpec((1, HALF), lambda i: (0, 0)),
      ],
      out_specs=pl.BlockSpec((G, HALF), lambda i: (0, 0)),
      out_shape=jax.ShapeDtypeStruct((G, G), _f32),
      scratch_shapes=[
          pltpu.VMEM((G, D), _f32),
          pltpu.VMEM((G, 1), _f32),
      ],
  )(s0, s1, a2d, b2d, batch3d, Wl1, bl1_2d, Wl2, bl2_2d)


def kernel(x, edge_index, edge_attr, batch,
           W0, b0, W1, b1, W2, b2, Wl1, bl1, Wl2, bl2):
  del edge_attr
  src2d = edge_index[0].reshape(EROWS, CHUNK)
  dst2d = edge_index[1].reshape(EROWS, CHUNK)
  batch3d = batch.reshape(N // ROWB, 1, ROWB)
  zeros_half = jnp.zeros((N, HALF), _f32)

  deg0, deg1 = _sc_degree(dst2d, zeros_half)
  u0, u1, a2d = _tc_first(x, deg0, deg1, W0)
  s0, s1 = _sc_propagate(u0, u1, src2d, dst2d)
  u0, u1 = _tc_mid(s0, s1, a2d, b0.reshape(1, D), W1)
  s0, s1 = _sc_propagate(u0, u1, src2d, dst2d)
  u0, u1 = _tc_mid(s0, s1, a2d, b1.reshape(1, D), W2)
  s0, s1 = _sc_propagate(u0, u1, src2d, dst2d)

  return _tc_final(s0, s1, a2d, b2.reshape(1, D), batch3d,
                   Wl1, bl1.reshape(1, HALF), Wl2, bl2.reshape(1, HALF))


# speedup vs baseline: 1.0188x; 1.0188x over previous
import jax
import jax.numpy as jnp
from jax import lax
from jax.experimental import pallas as pl
from jax.experimental.pallas import tpu as pltpu
from jax.experimental.pallas import tpu_sc as plsc

N = 10000
E = 160000
D = 256
G = 128
HALF = 128

NSUB = 16
CHUNK = 125
EROWS = E // CHUNK
TROWS = EROWS // NSUB
IGRP = 40
TROWS_D = EROWS // (2 * NSUB)
STRIPE = 624
TAILOFF = STRIPE * NSUB
TAIL = N - TAILOFF

_f32 = jnp.float32


def _mesh():
  return plsc.VectorSubcoreMesh(core_axis_name="c", subcore_axis_name="s")


def _stripe_copy(s, src_ref, dst_ref):
  base = pl.multiple_of(s * STRIPE, 8)
  pltpu.sync_copy(src_ref.at[pl.ds(base, STRIPE)], dst_ref.at[pl.ds(base, STRIPE)])

  @pl.when(s == 0)
  def _():
    pltpu.sync_copy(src_ref.at[pl.ds(TAILOFF, TAIL)],
                    dst_ref.at[pl.ds(TAILOFF, TAIL)])


def _prop_body(u0_hbm, u1_hbm, src_hbm, dst_hbm, out0_hbm, out1_hbm,
               acc_sh, sidx_v, didx_v, rows_a, rows_b, sem_a, sem_b):
  c = lax.axis_index("c")
  s = lax.axis_index("s")
  ib = pl.multiple_of(s * TROWS, 8)

  def per_core(fn):
    @pl.when(c == 0)
    def _():
      fn(u0_hbm, out0_hbm)

    @pl.when(c == 1)
    def _():
      fn(u1_hbm, out1_hbm)

  per_core(lambda u_hbm, out_hbm: _stripe_copy(s, u_hbm, acc_sh))
  plsc.subcore_barrier()

  def gather_start(k, rows_v, sem):
    per_core(lambda u_hbm, out_hbm:
             pltpu.async_copy(u_hbm.at[sidx_v.at[k]], rows_v, sem))

  def gather_wait(k, rows_v, sem):
    per_core(lambda u_hbm, out_hbm:
             pltpu.make_async_copy(u_hbm.at[sidx_v.at[k]], rows_v, sem).wait())

  @pl.loop(0, TROWS, step=IGRP)
  def _(jg):
    off = pl.multiple_of(ib + jg, 8)
    pltpu.sync_copy(src_hbm.at[pl.ds(off, IGRP)], sidx_v)
    pltpu.sync_copy(dst_hbm.at[pl.ds(off, IGRP)], didx_v)
    gather_start(0, rows_a, sem_a)

    @pl.loop(0, IGRP, step=2)
    def _(k):
      gather_wait(k, rows_a, sem_a)
      gather_start(k + 1, rows_b, sem_b)
      pltpu.sync_copy(rows_a, acc_sh.at[didx_v.at[k]], add=True)
      gather_wait(k + 1, rows_b, sem_b)

      @pl.when(k + 2 < IGRP)
      def _():
        gather_start(k + 2, rows_a, sem_a)

      pltpu.sync_copy(rows_b, acc_sh.at[didx_v.at[k + 1]], add=True)

  plsc.subcore_barrier()
  per_core(lambda u_hbm, out_hbm: _stripe_copy(s, acc_sh, out_hbm))


@jax.jit
def _sc_propagate(u0, u1, src2d, dst2d):
  run = pl.kernel(
      _prop_body,
      out_type=(jax.ShapeDtypeStruct((N, HALF), _f32),
                jax.ShapeDtypeStruct((N, HALF), _f32)),
      mesh=_mesh(),
      scratch_types=[
          pltpu.VMEM_SHARED((N, HALF), _f32),
          pltpu.VMEM((IGRP, CHUNK), jnp.int32),
          pltpu.VMEM((IGRP, CHUNK), jnp.int32),
          pltpu.VMEM((CHUNK, HALF), _f32),
          pltpu.VMEM((CHUNK, HALF), _f32),
          pltpu.SemaphoreType.DMA,
          pltpu.SemaphoreType.DMA,
      ],
  )
  return run(u0, u1, src2d, dst2d)


def _deg_body(dst_hbm, zeros_hbm, deg0_hbm, deg1_hbm, acc_sh, didx_v, ones_v,
              sem_a, sem_b):
  c = lax.axis_index("c")
  s = lax.axis_index("s")
  ib = pl.multiple_of(c * (EROWS // 2) + s * TROWS_D, 8)
  pltpu.sync_copy(dst_hbm.at[pl.ds(ib, TROWS_D)], didx_v)

  @pl.loop(0, CHUNK)
  def _(r):
    @pl.loop(0, HALF, step=16)
    def _(q):
      ones_v[r, pl.ds(q, 16)] = jnp.full((16,), 1.0, _f32)

  _stripe_copy(s, zeros_hbm, acc_sh)
  plsc.subcore_barrier()

  def deg_scat_start(j, sem):
    pltpu.async_copy(ones_v, acc_sh.at[didx_v.at[j]], sem, add=True)

  def deg_scat_wait(j, sem):
    pltpu.make_async_copy(ones_v, acc_sh.at[didx_v.at[j]], sem).wait()

  deg_scat_start(0, sem_a)
  deg_scat_start(1, sem_b)

  @pl.loop(2, TROWS_D, step=2)
  def _(j):
    deg_scat_wait(j, sem_a)
    deg_scat_start(j, sem_a)
    deg_scat_wait(j + 1, sem_b)
    deg_scat_start(j + 1, sem_b)

  deg_scat_wait(0, sem_a)
  deg_scat_wait(1, sem_b)
  plsc.subcore_barrier()

  @pl.when(c == 0)
  def _():
    _stripe_copy(s, acc_sh, deg0_hbm)

  @pl.when(c == 1)
  def _():
    _stripe_copy(s, acc_sh, deg1_hbm)


@jax.jit
def _sc_degree(dst2d, zeros_half):
  run = pl.kernel(
      _deg_body,
      out_type=(jax.ShapeDtypeStruct((N, HALF), _f32),
                jax.ShapeDtypeStruct((N, HALF), _f32)),
      mesh=_mesh(),
      scratch_types=[
          pltpu.VMEM_SHARED((N, HALF), _f32),
          pltpu.VMEM((TROWS_D, CHUNK), jnp.int32),
          pltpu.VMEM((CHUNK, HALF), _f32),
          pltpu.SemaphoreType.DMA,
          pltpu.SemaphoreType.DMA,
      ],
  )
  return run(dst2d, zeros_half)


ROWB = 2000


def _tc_z0_body(x_ref, w_ref, z_ref):
  z_ref[...] = jnp.dot(x_ref[...].astype(jnp.bfloat16),
                       w_ref[...].astype(jnp.bfloat16),
                       preferred_element_type=_f32)


def _tc_scale_body(z_ref, d_ref, d2_ref, u0_ref, u1_ref, a_ref):
  a = lax.rsqrt(d_ref[:, 0:1] + d2_ref[:, 0:1] + 1.0)
  a_ref[...] = a
  u = a * z_ref[...]
  u0_ref[...] = u[:, :HALF]
  u1_ref[...] = u[:, HALF:]


def _tc_mid_body(s0_ref, s1_ref, a_ref, b_ref, w_ref, u0_ref, u1_ref):
  a = a_ref[...]
  h = jnp.concatenate([s0_ref[...], s1_ref[...]], axis=1)
  h = jnp.maximum(a * h + b_ref[...], 0.0)
  u = a * jnp.dot(h.astype(jnp.bfloat16), w_ref[...].astype(jnp.bfloat16),
                  preferred_element_type=_f32)
  u0_ref[...] = u[:, :HALF]
  u1_ref[...] = u[:, HALF:]


def _tc_final_body(s0_ref, s1_ref, a_ref, b_ref, batch_ref,
                   wl1_ref, bl1_ref, wl2_ref, bl2_ref, out_ref,
                   pooled_sc, cnt_sc):
  i = pl.program_id(0)

  @pl.when(i == 0)
  def _():
    pooled_sc[...] = jnp.zeros_like(pooled_sc)
    cnt_sc[...] = jnp.zeros_like(cnt_sc)

  a = a_ref[...]
  h = jnp.concatenate([s0_ref[...], s1_ref[...]], axis=1)
  h = jnp.maximum(a * h + b_ref[...], 0.0)
  g = batch_ref[0, 0]
  oh = (g[:, None] == lax.broadcasted_iota(jnp.int32, (ROWB, G), 1))
  oh = oh.astype(_f32)
  pooled_sc[...] += lax.dot_general(
      oh, h, (((0,), (0,)), ((), ())), preferred_element_type=_f32)
  cnt_sc[...] += lax.dot_general(
      oh, jnp.ones((ROWB, 1), _f32), (((0,), (0,)), ((), ())),
      preferred_element_type=_f32)

  @pl.when(i == pl.num_programs(0) - 1)
  def _():
    pooled = pooled_sc[...] / jnp.maximum(cnt_sc[...], 1.0)
    p1 = jnp.maximum(
        jnp.dot(pooled, wl1_ref[...], preferred_element_type=_f32)
        + bl1_ref[...], 0.0)
    out_ref[...] = (jnp.dot(p1, wl2_ref[...], preferred_element_type=_f32)
                    + bl2_ref[...])


def _row_specs():
  return [
      pl.BlockSpec((ROWB, HALF), lambda i: (i, 0)),
      pl.BlockSpec((ROWB, HALF), lambda i: (i, 0)),
      pl.BlockSpec((ROWB, 1), lambda i: (i, 0)),
      pl.BlockSpec((1, D), lambda i: (0, 0)),
  ]


@jax.jit
def _tc_z0(x, W):
  return pl.pallas_call(
      _tc_z0_body,
      grid=(N // ROWB,),
      in_specs=[
          pl.BlockSpec((ROWB, D), lambda i: (i, 0)),
          pl.BlockSpec((D, D), lambda i: (0, 0)),
      ],
      out_specs=pl.BlockSpec((ROWB, D), lambda i: (i, 0)),
      out_shape=jax.ShapeDtypeStruct((N, D), _f32),
  )(x, W)


@jax.jit
def _tc_scale(z, deg0, deg1):
  return pl.pallas_call(
      _tc_scale_body,
      grid=(N // ROWB,),
      in_specs=[
          pl.BlockSpec((ROWB, D), lambda i: (i, 0)),
          pl.BlockSpec((ROWB, HALF), lambda i: (i, 0)),
          pl.BlockSpec((ROWB, HALF), lambda i: (i, 0)),
      ],
      out_specs=[pl.BlockSpec((ROWB, HALF), lambda i: (i, 0))] * 2
      + [pl.BlockSpec((ROWB, 1), lambda i: (i, 0))],
      out_shape=[jax.ShapeDtypeStruct((N, HALF), _f32)] * 2
      + [jax.ShapeDtypeStruct((N, 1), _f32)],
  )(z, deg0, deg1)


@jax.jit
def _tc_mid(s0, s1, a2d, b2d, W):
  return pl.pallas_call(
      _tc_mid_body,
      grid=(N // ROWB,),
      in_specs=_row_specs() + [pl.BlockSpec((D, D), lambda i: (0, 0))],
      out_specs=[pl.BlockSpec((ROWB, HALF), lambda i: (i, 0))] * 2,
      out_shape=[jax.ShapeDtypeStruct((N, HALF), _f32)] * 2,
  )(s0, s1, a2d, b2d, W)


@jax.jit
def _tc_final(s0, s1, a2d, b2d, batch3d, Wl1, bl1_2d, Wl2, bl2_2d):
  return pl.pallas_call(
      _tc_final_body,
      grid=(N // ROWB,),
      in_specs=_row_specs() + [
          pl.BlockSpec((1, 1, ROWB), lambda i: (i, 0, 0)),
          pl.BlockSpec((D, HALF), lambda i: (0, 0)),
          pl.BlockSpec((1, HALF), lambda i: (0, 0)),
          pl.BlockSpec((HALF, HALF), lambda i: (0, 0)),
          pl.BlockSpec((1, HALF), lambda i: (0, 0)),
      ],
      out_specs=pl.BlockSpec((G, HALF), lambda i: (0, 0)),
      out_shape=jax.ShapeDtypeStruct((G, G), _f32),
      scratch_shapes=[
          pltpu.VMEM((G, D), _f32),
          pltpu.VMEM((G, 1), _f32),
      ],
  )(s0, s1, a2d, b2d, batch3d, Wl1, bl1_2d, Wl2, bl2_2d)


def kernel(x, edge_index, edge_attr, batch,
           W0, b0, W1, b1, W2, b2, Wl1, bl1, Wl2, bl2):
  del edge_attr
  src2d = edge_index[0].reshape(EROWS, CHUNK)
  dst2d = edge_index[1].reshape(EROWS, CHUNK)
  batch3d = batch.reshape(N // ROWB, 1, ROWB)
  zeros_half = jnp.zeros((N, HALF), _f32)

  deg0, deg1 = _sc_degree(dst2d, zeros_half)
  z0 = _tc_z0(x, W0)
  u0, u1, a2d = _tc_scale(z0, deg0, deg1)
  s0, s1 = _sc_propagate(u0, u1, src2d, dst2d)
  u0, u1 = _tc_mid(s0, s1, a2d, b0.reshape(1, D), W1)
  s0, s1 = _sc_propagate(u0, u1, src2d, dst2d)
  u0, u1 = _tc_mid(s0, s1, a2d, b1.reshape(1, D), W2)
  s0, s1 = _sc_propagate(u0, u1, src2d, dst2d)

  return _tc_final(s0, s1, a2d, b2.reshape(1, D), batch3d,
                   Wl1, bl1.reshape(1, HALF), Wl2, bl2.reshape(1, HALF))
